# baseline (speedup 1.0000x reference)
import functools

import jax
import jax.numpy as jnp
from jax import lax
from jax.experimental import pallas as pl
from jax.experimental.pallas import tpu as pltpu
from jax.experimental.pallas import tpu_sc as plsc

NUM_USERS = 1000000
NUM_ITEMS = 100000
EMB_DIM = 64
BATCH = 16384

NC, NS, L = 2, 16, 16
NW = NC * NS
BPW = BATCH // NW
CHUNK = 128
NCHUNK = BPW // CHUNK
NGROUP = BPW // L


def _body(user_hbm, item_hbm, uemb_hbm, iemb_hbm, ub_hbm, ib_hbm, gb_hbm,
          out_hbm,
          uidx_v, iidx_v, urows_v, irows_v, ub_v, ib_v, gb_v, out_v, sem):
    wid = lax.axis_index("s") * NC + lax.axis_index("c")
    base = wid * BPW

    pltpu.sync_copy(user_hbm.at[pl.ds(wid * NCHUNK, NCHUNK)], uidx_v)
    pltpu.sync_copy(item_hbm.at[pl.ds(wid * NCHUNK, NCHUNK)], iidx_v)
    pltpu.sync_copy(gb_hbm, gb_v)

    copies = []
    for c in range(NCHUNK):
        sl = pl.ds(c * CHUNK, CHUNK)
        copies.append(pltpu.async_copy(uemb_hbm.at[uidx_v.at[c]],
                                       urows_v.at[sl], sem))
        copies.append(pltpu.async_copy(iemb_hbm.at[iidx_v.at[c]],
                                       irows_v.at[sl], sem))
        copies.append(pltpu.async_copy(ub_hbm.at[uidx_v.at[c]],
                                       ub_v.at[sl], sem))
        copies.append(pltpu.async_copy(ib_hbm.at[iidx_v.at[c]],
                                       ib_v.at[sl], sem))
    for cp in copies:
        cp.wait()

    gb = gb_v[...]

    def group(g, _):
        e0 = g * L
        row_idx = e0 + lax.iota(jnp.int32, L)
        acc = ub_v[pl.ds(e0, L)] + ib_v[pl.ds(e0, L)] + gb
        for d in range(EMB_DIM):
            col = jnp.full((L,), d, jnp.int32)
            ucol = plsc.load_gather(urows_v, [row_idx, col])
            icol = plsc.load_gather(irows_v, [row_idx, col])
            acc = acc + ucol * icol
        out_v[pl.ds(e0, L)] = acc
        return _

    lax.fori_loop(0, NGROUP, group, None)
    pltpu.sync_copy(out_v, out_hbm.at[pl.ds(base, BPW)])


@functools.partial(jax.jit, static_argnames=())
def kernel(user, item, user_emb, item_emb, user_bias, item_bias, global_bias):
    user2d = user.reshape(NW * NCHUNK, CHUNK)
    item2d = item.reshape(NW * NCHUNK, CHUNK)
    ub1d = user_bias.reshape(NUM_USERS)
    ib1d = item_bias.reshape(NUM_ITEMS)
    gb16 = jnp.broadcast_to(global_bias, (L,))

    run = pl.kernel(
        _body,
        out_type=jax.ShapeDtypeStruct((BATCH,), jnp.float32),
        mesh=plsc.VectorSubcoreMesh(core_axis_name="c", subcore_axis_name="s",
                                    num_cores=NC, num_subcores=NS),
        scratch_types=[
            pltpu.VMEM((NCHUNK, CHUNK), jnp.int32),
            pltpu.VMEM((NCHUNK, CHUNK), jnp.int32),
            pltpu.VMEM((BPW, EMB_DIM), jnp.float32),
            pltpu.VMEM((BPW, EMB_DIM), jnp.float32),
            pltpu.VMEM((BPW,), jnp.float32),
            pltpu.VMEM((BPW,), jnp.float32),
            pltpu.VMEM((L,), jnp.float32),
            pltpu.VMEM((BPW,), jnp.float32),
            pltpu.SemaphoreType.DMA,
        ],
        compiler_params=pltpu.CompilerParams(needs_layout_passes=False,
                                             use_tc_tiling_on_sc=False),
    )
    return run(user2d, item2d, user_emb, item_emb, ub1d, ib1d, gb16)
